# Initial kernel scaffold; baseline (speedup 1.0000x reference)
import jax
import jax.numpy as jnp
from jax.experimental import pallas as pl


def kernel(x, index, dim_size):
    raise NotImplementedError("write your pallas kernel here")


# speedup vs baseline: 5.2702x; 5.2702x over previous
import functools

import jax
import jax.numpy as jnp
from jax import lax
from jax.experimental import pallas as pl
from jax.experimental.pallas import tpu as pltpu
from jax.experimental.pallas import tpu_sc as plsc

E = 320000
D = 128
N = 10000
NC = 2
NS = 16
NW = NC * NS
EW = E // NW
B = 80
K = EW // B
RZ = 640
NP = NS * RZ
CW = 16

_mesh = plsc.VectorSubcoreMesh(
    core_axis_name="c", subcore_axis_name="s", num_cores=NC, num_subcores=NS
)


@functools.partial(
    pl.kernel,
    out_type=(
        jax.ShapeDtypeStruct((NC, NP, D), jnp.float32),
        jax.ShapeDtypeStruct((NC, NP, CW), jnp.float32),
    ),
    mesh=_mesh,
    compiler_params=pltpu.CompilerParams(use_tc_tiling_on_sc=False),
    scratch_types=[
        pltpu.VMEM((K, B), jnp.int32),
        pltpu.VMEM((B, D), jnp.float32),
        pltpu.VMEM((B, CW), jnp.float32),
        pltpu.VMEM_SHARED((NP, D), jnp.float32),
        pltpu.VMEM_SHARED((NP, CW), jnp.float32),
    ],
)
def _sc_partials(x_hbm, idx_hbm, ones_hbm, zs_hbm, zc_hbm,
                 ps_hbm, pc_hbm, idxv, xbuf, obuf, acc, cnt):
    c = lax.axis_index("c")
    s = lax.axis_index("s")
    wid = c * NS + s
    pltpu.sync_copy(zs_hbm, xbuf)
    pltpu.sync_copy(zc_hbm, obuf)
    for j in range(RZ // B):
        pltpu.sync_copy(xbuf, acc.at[pl.ds(s * RZ + j * B, B)])
        pltpu.sync_copy(obuf, cnt.at[pl.ds(s * RZ + j * B, B)])
    pltpu.sync_copy(idx_hbm.at[wid], idxv)
    pltpu.sync_copy(ones_hbm, obuf)
    plsc.subcore_barrier()

    ebase = wid * EW

    def step(k, carry):
        pltpu.sync_copy(x_hbm.at[pl.ds(ebase + k * B, B)], xbuf)
        pltpu.sync_copy(xbuf, acc.at[idxv.at[k]], add=True)
        pltpu.sync_copy(obuf, cnt.at[idxv.at[k]], add=True)
        return carry

    lax.fori_loop(0, K, step, 0)
    plsc.subcore_barrier()
    for j in range(RZ // B):
        pltpu.sync_copy(acc.at[pl.ds(s * RZ + j * B, B)], xbuf)
        pltpu.sync_copy(xbuf, ps_hbm.at[c, pl.ds(s * RZ + j * B, B)])
        pltpu.sync_copy(cnt.at[pl.ds(s * RZ + j * B, B)], obuf)
        pltpu.sync_copy(obuf, pc_hbm.at[c, pl.ds(s * RZ + j * B, B)])


ROWS_BLK = 640


def _combine_body(ps_ref, pc_ref, o_ref):
    ssum = ps_ref[0] + ps_ref[1]
    csum = pc_ref[0] + pc_ref[1]
    o_ref[...] = ssum / jnp.maximum(csum[:, 0:1], 1.0)


_combine = pl.pallas_call(
    _combine_body,
    grid=(NP // ROWS_BLK,),
    in_specs=[
        pl.BlockSpec((NC, ROWS_BLK, D), lambda i: (0, i, 0)),
        pl.BlockSpec((NC, ROWS_BLK, CW), lambda i: (0, i, 0)),
    ],
    out_specs=pl.BlockSpec((ROWS_BLK, D), lambda i: (i, 0)),
    out_shape=jax.ShapeDtypeStruct((NP, D), jnp.float32),
)


def kernel(x, index, dim_size):
    del dim_size
    idx3d = index.astype(jnp.int32).reshape(NW, K, B)
    ones = jnp.ones((B, CW), jnp.float32)
    zs = jnp.zeros((B, D), jnp.float32)
    zc = jnp.zeros((B, CW), jnp.float32)
    ps, pc = _sc_partials(x, idx3d, ones, zs, zc)
    return _combine(ps, pc)[:N]


# speedup vs baseline: 7.9141x; 1.5017x over previous
import functools

import jax
import jax.numpy as jnp
from jax import lax
from jax.experimental import pallas as pl
from jax.experimental.pallas import tpu as pltpu
from jax.experimental.pallas import tpu_sc as plsc

E = 320000
D = 128
N = 10000
NC = 2
NS = 16
NW = NC * NS
EW = E // NW
B = 100
K = EW // B
RZ = 640
NP = NS * RZ
CW = 16
ZB = 64

_mesh = plsc.VectorSubcoreMesh(
    core_axis_name="c", subcore_axis_name="s", num_cores=NC, num_subcores=NS
)


@functools.partial(
    pl.kernel,
    out_type=(
        jax.ShapeDtypeStruct((NC, NP, D), jnp.float32),
        jax.ShapeDtypeStruct((NC, NP, CW), jnp.float32),
    ),
    mesh=_mesh,
    compiler_params=pltpu.CompilerParams(use_tc_tiling_on_sc=False),
    scratch_types=[
        pltpu.VMEM((K, B), jnp.int32),
        pltpu.VMEM((B, D), jnp.float32),
        pltpu.VMEM((B, D), jnp.float32),
        pltpu.VMEM((B, CW), jnp.float32),
        pltpu.SemaphoreType.DMA,
        pltpu.SemaphoreType.DMA,
        pltpu.VMEM_SHARED((NP, D), jnp.float32),
        pltpu.VMEM_SHARED((NP, CW), jnp.float32),
    ],
)
def _sc_partials(x_hbm, idx_hbm, ones_hbm, zs_hbm, zc_hbm,
                 ps_hbm, pc_hbm, idxv, xbuf0, xbuf1, obuf, sem0, sem1,
                 acc, cnt):
    c = lax.axis_index("c")
    s = lax.axis_index("s")
    wid = c * NS + s
    pltpu.sync_copy(zs_hbm, xbuf0.at[pl.ds(0, ZB)])
    pltpu.sync_copy(zc_hbm, obuf.at[pl.ds(0, ZB)])
    for j in range(RZ // ZB):
        pltpu.sync_copy(xbuf0.at[pl.ds(0, ZB)], acc.at[pl.ds(s * RZ + j * ZB, ZB)])
        pltpu.sync_copy(obuf.at[pl.ds(0, ZB)], cnt.at[pl.ds(s * RZ + j * ZB, ZB)])
    pltpu.sync_copy(idx_hbm.at[wid], idxv)
    pltpu.sync_copy(ones_hbm, obuf)
    plsc.subcore_barrier()

    ebase = wid * EW
    bufs = (xbuf0, xbuf1)
    sems = (sem0, sem1)

    def load(k, b):
        return pltpu.async_copy(x_hbm.at[pl.ds(ebase + k * B, B)], bufs[b],
                                sems[b])

    def wait(k, b):
        pltpu.make_async_copy(x_hbm.at[pl.ds(ebase + k * B, B)], bufs[b],
                              sems[b]).wait()

    def process(k, b):
        wait(k, b)
        pltpu.sync_copy(bufs[b], acc.at[idxv.at[k]], add=True)
        pltpu.sync_copy(obuf, cnt.at[idxv.at[k]], add=True)

    load(0, 0)
    load(1, 1)

    def group(g, carry):
        for b in range(2):
            k = 2 * g + b
            process(k, b)
            load(k + 2, b)
        return carry

    lax.fori_loop(0, K // 2 - 1, group, 0)
    process(K - 2, 0)
    process(K - 1, 1)

    plsc.subcore_barrier()
    for j in range(RZ // ZB):
        pltpu.sync_copy(acc.at[pl.ds(s * RZ + j * ZB, ZB)], xbuf0.at[pl.ds(0, ZB)])
        pltpu.sync_copy(xbuf0.at[pl.ds(0, ZB)], ps_hbm.at[c, pl.ds(s * RZ + j * ZB, ZB)])
        pltpu.sync_copy(cnt.at[pl.ds(s * RZ + j * ZB, ZB)], obuf.at[pl.ds(0, ZB)])
        pltpu.sync_copy(obuf.at[pl.ds(0, ZB)], pc_hbm.at[c, pl.ds(s * RZ + j * ZB, ZB)])


ROWS_BLK = 640


def _combine_body(ps_ref, pc_ref, o_ref):
    ssum = ps_ref[0] + ps_ref[1]
    csum = pc_ref[0] + pc_ref[1]
    o_ref[...] = ssum / jnp.maximum(csum[:, 0:1], 1.0)


_combine = pl.pallas_call(
    _combine_body,
    grid=(NP // ROWS_BLK,),
    in_specs=[
        pl.BlockSpec((NC, ROWS_BLK, D), lambda i: (0, i, 0)),
        pl.BlockSpec((NC, ROWS_BLK, CW), lambda i: (0, i, 0)),
    ],
    out_specs=pl.BlockSpec((ROWS_BLK, D), lambda i: (i, 0)),
    out_shape=jax.ShapeDtypeStruct((NP, D), jnp.float32),
)


def kernel(x, index, dim_size):
    del dim_size
    idx3d = index.astype(jnp.int32).reshape(NW, K, B)
    ones = jnp.ones((B, CW), jnp.float32)
    zs = jnp.zeros((ZB, D), jnp.float32)
    zc = jnp.zeros((ZB, CW), jnp.float32)
    ps, pc = _sc_partials(x, idx3d, ones, zs, zc)
    return _combine(ps, pc)[:N]
